# Initial kernel scaffold; baseline (speedup 1.0000x reference)
import jax
import jax.numpy as jnp
from jax.experimental import pallas as pl


def kernel(feat, edge_index, etypes, weight):
    raise NotImplementedError("write your pallas kernel here")


# speedup vs baseline: 5.4281x; 5.4281x over previous
import functools

import jax
import jax.numpy as jnp
from jax import lax
from jax.experimental import pallas as pl
from jax.experimental.pallas import tpu as pltpu
from jax.experimental.pallas import tpu_sc as plsc

N_NODES = 10000
D = 128
R = 8
N_EDGES = 160000

NC = 2
NS = 16
NW = NC * NS
BATCH = 128
E_PAD = 163840
EPW = E_PAD // NW
NB = EPW // BATCH
N_PAD = 10240
SLAB = N_PAD // NS
SLAB_CP = SLAB // BATCH


def _relmm_body(f_ref, w_ref, o_ref):
    o_ref[0] = jnp.dot(f_ref[...], w_ref[0], preferred_element_type=jnp.float32)


def _rel_matmul(feat, weight):
    bn = 1000
    return pl.pallas_call(
        _relmm_body,
        grid=(N_NODES // bn, R),
        in_specs=[
            pl.BlockSpec((bn, D), lambda n, r: (n, 0)),
            pl.BlockSpec((1, D, D), lambda n, r: (r, 0, 0)),
        ],
        out_specs=pl.BlockSpec((1, bn, D), lambda n, r: (r, n, 0)),
        out_shape=jax.ShapeDtypeStruct((R, N_NODES, D), jnp.float32),
    )(feat, weight)


def _add_body(p_ref, o_ref):
    o_ref[...] = p_ref[0] + p_ref[1]


def _sum_partials(partials):
    bn = 1280
    return pl.pallas_call(
        _add_body,
        grid=(N_PAD // bn,),
        in_specs=[pl.BlockSpec((2, bn, D), lambda i: (0, i, 0))],
        out_specs=pl.BlockSpec((bn, D), lambda i: (i, 0)),
        out_shape=jax.ShapeDtypeStruct((N_PAD, D), jnp.float32),
    )(partials)


def _sc_gather_scatter(f_table, src_w, et_w, dst_w):
    mesh = plsc.VectorSubcoreMesh(core_axis_name="c", subcore_axis_name="s")

    @functools.partial(
        pl.kernel,
        mesh=mesh,
        out_type=jax.ShapeDtypeStruct((NC, N_PAD, D), jnp.float32),
        scratch_types=[
            pltpu.VMEM((EPW,), jnp.int32),
            pltpu.VMEM((EPW,), jnp.int32),
            pltpu.VMEM((NB, BATCH), jnp.int32),
            pltpu.VMEM((BATCH, D), jnp.float32),
            pltpu.VMEM_SHARED((N_PAD, D), jnp.float32),
            pltpu.SemaphoreType.DMA,
        ],
    )
    def sc_kern(f_hbm, src_hbm, et_hbm, dst_hbm, out_hbm,
                src_v, key_v, dst_v, rows_v, acc, sem):
        cid = lax.axis_index("c")
        sid = lax.axis_index("s")
        wid = cid * NS + sid

        pltpu.sync_copy(src_hbm.at[wid], src_v)
        pltpu.sync_copy(et_hbm.at[wid], key_v)
        pltpu.sync_copy(dst_hbm.at[wid], dst_v)

        zero16 = jnp.zeros((16,), jnp.float32)

        def zero_body(i, _):
            for c in range(D // 16):
                rows_v[i, pl.ds(c * 16, 16)] = zero16
            return _

        lax.fori_loop(0, BATCH, zero_body, None)
        for k in range(SLAB_CP):
            pltpu.sync_copy(rows_v, acc.at[pl.ds(sid * SLAB + k * BATCH, BATCH)])

        def key_body(i, _):
            sl = pl.ds(i * 16, 16)
            key_v[sl] = key_v[sl] * N_NODES + src_v[sl]
            return _

        lax.fori_loop(0, EPW // 16, key_body, None)

        plsc.subcore_barrier()

        def edge_body(j, _):
            pltpu.async_copy(
                f_hbm.at[key_v.at[pl.ds(j * BATCH, BATCH)]], rows_v, sem
            ).wait()
            pltpu.sync_copy(rows_v, acc.at[dst_v.at[j]], add=True)
            return _

        lax.fori_loop(0, NB, edge_body, None)

        plsc.subcore_barrier()
        pltpu.sync_copy(acc.at[pl.ds(sid * SLAB, SLAB)],
                        out_hbm.at[cid, pl.ds(sid * SLAB, SLAB)])

    def wrapped(f_table, src_w, et_w, dst_w):
        return sc_kern(f_table, src_w, et_w, dst_w)

    return wrapped(f_table, src_w, et_w, dst_w)


def kernel(feat, edge_index, etypes, weight):
    src = edge_index[0]
    dst = edge_index[1]
    pad = E_PAD - N_EDGES
    src_p = jnp.concatenate([src, jnp.zeros((pad,), jnp.int32)])
    et_p = jnp.concatenate([etypes, jnp.zeros((pad,), jnp.int32)])
    dst_p = jnp.concatenate(
        [dst, N_NODES + (jnp.arange(pad, dtype=jnp.int32) % (N_PAD - N_NODES))]
    )

    f_table = _rel_matmul(feat, weight).reshape(R * N_NODES, D)
    partials = _sc_gather_scatter(
        f_table,
        src_p.reshape(NW, EPW),
        et_p.reshape(NW, EPW),
        dst_p.reshape(NW, NB, BATCH),
    )
    out = _sum_partials(partials)
    return out[:N_NODES]


# speedup vs baseline: 5.9004x; 1.0870x over previous
import functools

import jax
import jax.numpy as jnp
from jax import lax
from jax.experimental import pallas as pl
from jax.experimental.pallas import tpu as pltpu
from jax.experimental.pallas import tpu_sc as plsc

N_NODES = 10000
D = 128
R = 8
N_EDGES = 160000

NC = 2
NS = 16
NW = NC * NS
BATCH = 128
E_PAD = 163840
EPW = E_PAD // NW
NB = EPW // BATCH
N_PAD = 10240
SLAB = N_PAD // NS
SLAB_CP = SLAB // BATCH


def _relmm_body(f_ref, w_ref, o_ref):
    o_ref[0] = jnp.dot(f_ref[...], w_ref[0], preferred_element_type=jnp.float32)


def _rel_matmul(feat, weight):
    bn = 1000
    return pl.pallas_call(
        _relmm_body,
        grid=(N_NODES // bn, R),
        in_specs=[
            pl.BlockSpec((bn, D), lambda n, r: (n, 0)),
            pl.BlockSpec((1, D, D), lambda n, r: (r, 0, 0)),
        ],
        out_specs=pl.BlockSpec((1, bn, D), lambda n, r: (r, n, 0)),
        out_shape=jax.ShapeDtypeStruct((R, N_NODES, D), jnp.float32),
    )(feat, weight)


def _add_body(p_ref, o_ref):
    o_ref[...] = p_ref[0] + p_ref[1]


def _sum_partials(partials):
    bn = 1280
    return pl.pallas_call(
        _add_body,
        grid=(N_PAD // bn,),
        in_specs=[pl.BlockSpec((2, bn, D), lambda i: (0, i, 0))],
        out_specs=pl.BlockSpec((bn, D), lambda i: (i, 0)),
        out_shape=jax.ShapeDtypeStruct((N_PAD, D), jnp.float32),
    )(partials)


def _sc_gather_scatter(f_table, src_w, et_w, dst_w):
    mesh = plsc.VectorSubcoreMesh(core_axis_name="c", subcore_axis_name="s")

    @functools.partial(
        pl.kernel,
        mesh=mesh,
        out_type=jax.ShapeDtypeStruct((NC, N_PAD, D), jnp.float32),
        scratch_types=[
            pltpu.VMEM((EPW,), jnp.int32),
            pltpu.VMEM((EPW,), jnp.int32),
            pltpu.VMEM((NB, BATCH), jnp.int32),
            pltpu.VMEM((BATCH, D), jnp.float32),
            pltpu.VMEM((BATCH, D), jnp.float32),
            pltpu.VMEM_SHARED((N_PAD, D), jnp.float32),
            pltpu.SemaphoreType.DMA,
            pltpu.SemaphoreType.DMA,
        ],
    )
    def sc_kern(f_hbm, src_hbm, et_hbm, dst_hbm, out_hbm,
                src_v, key_v, dst_v, rows0, rows1, acc,
                sem0, sem1):
        cid = lax.axis_index("c")
        sid = lax.axis_index("s")
        wid = cid * NS + sid

        pltpu.sync_copy(src_hbm.at[wid], src_v)
        pltpu.sync_copy(et_hbm.at[wid], key_v)
        pltpu.sync_copy(dst_hbm.at[wid], dst_v)

        zero16 = jnp.zeros((16,), jnp.float32)

        def zero_body(i, _):
            for c in range(D // 16):
                rows0[i, pl.ds(c * 16, 16)] = zero16
            return _

        lax.fori_loop(0, BATCH, zero_body, None)
        for k in range(SLAB_CP):
            pltpu.sync_copy(rows0, acc.at[pl.ds(sid * SLAB + k * BATCH, BATCH)])

        def key_body(i, _):
            sl = pl.ds(i * 16, 16)
            key_v[sl] = key_v[sl] * N_NODES + src_v[sl]
            return _

        lax.fori_loop(0, EPW // 16, key_body, None)

        plsc.subcore_barrier()

        bufs = (rows0, rows1)
        sems = (sem0, sem1)
        NBUF = 2

        def gstart(idx, b):
            pltpu.async_copy(
                f_hbm.at[key_v.at[pl.ds(idx * BATCH, BATCH)]], bufs[b], sems[b]
            )

        for b in range(NBUF):
            gstart(b, b)

        def pipe_body(j, _):
            for b in range(NBUF):
                idx = j * NBUF + b
                pltpu.make_async_copy(
                    f_hbm.at[pl.ds(0, BATCH)], bufs[b], sems[b]
                ).wait()
                pltpu.sync_copy(bufs[b], acc.at[dst_v.at[idx]], add=True)

                @pl.when(idx + NBUF < NB)
                def _start_next():
                    gstart(idx + NBUF, b)
            return _

        lax.fori_loop(0, NB // NBUF, pipe_body, None)

        plsc.subcore_barrier()
        pltpu.sync_copy(acc.at[pl.ds(sid * SLAB, SLAB)],
                        out_hbm.at[cid, pl.ds(sid * SLAB, SLAB)])

    def wrapped(f_table, src_w, et_w, dst_w):
        return sc_kern(f_table, src_w, et_w, dst_w)

    return wrapped(f_table, src_w, et_w, dst_w)


def kernel(feat, edge_index, etypes, weight):
    src = edge_index[0]
    dst = edge_index[1]
    pad = E_PAD - N_EDGES
    src_p = jnp.concatenate([src, jnp.zeros((pad,), jnp.int32)])
    et_p = jnp.concatenate([etypes, jnp.zeros((pad,), jnp.int32)])
    dst_p = jnp.concatenate(
        [dst, N_NODES + (jnp.arange(pad, dtype=jnp.int32) % (N_PAD - N_NODES))]
    )

    f_table = _rel_matmul(feat, weight).reshape(R * N_NODES, D)
    partials = _sc_gather_scatter(
        f_table,
        src_p.reshape(NW, EPW),
        et_p.reshape(NW, EPW),
        dst_p.reshape(NW, NB, BATCH),
    )
    out = _sum_partials(partials)
    return out[:N_NODES]


# speedup vs baseline: 6.5862x; 1.1162x over previous
import functools

import jax
import jax.numpy as jnp
from jax import lax
from jax.experimental import pallas as pl
from jax.experimental.pallas import tpu as pltpu
from jax.experimental.pallas import tpu_sc as plsc

N_NODES = 10000
D = 128
R = 8
N_EDGES = 160000

NC = 2
NS = 16
BATCH = 128
NB0 = 64
NB1 = 16
NBMAX = NB0
NBT = NS * (NB0 + NB1)
E_PAD = NBT * BATCH
N_PAD = 10112
SLAB = N_PAD // NS
NBUF = 2


def _relmm_body(f_ref, w_ref, o_ref):
    o_ref[0] = jnp.dot(f_ref[...], w_ref[0], preferred_element_type=jnp.float32)


def _rel_matmul(feat, weight):
    bn = 1000
    return pl.pallas_call(
        _relmm_body,
        grid=(N_NODES // bn, R),
        in_specs=[
            pl.BlockSpec((bn, D), lambda n, r: (n, 0)),
            pl.BlockSpec((1, D, D), lambda n, r: (r, 0, 0)),
        ],
        out_specs=pl.BlockSpec((1, bn, D), lambda n, r: (r, n, 0)),
        out_shape=jax.ShapeDtypeStruct((R, N_NODES, D), jnp.float32),
    )(feat, weight)


def _add_body(p_ref, o_ref):
    o_ref[...] = p_ref[0] + p_ref[1]


def _sum_partials(partials):
    bn = 1264
    return pl.pallas_call(
        _add_body,
        grid=(N_PAD // bn,),
        in_specs=[pl.BlockSpec((2, bn, D), lambda i: (0, i, 0))],
        out_specs=pl.BlockSpec((bn, D), lambda i: (i, 0)),
        out_shape=jax.ShapeDtypeStruct((N_PAD, D), jnp.float32),
    )(partials)


def _sc_gather_scatter(f_table, src_w, et_w, dst_w):
    mesh = plsc.VectorSubcoreMesh(core_axis_name="c", subcore_axis_name="s")

    @functools.partial(
        pl.kernel,
        mesh=mesh,
        out_type=jax.ShapeDtypeStruct((NC, N_PAD, D), jnp.float32),
        scratch_types=[
            pltpu.VMEM((NBMAX, BATCH), jnp.int32),
            pltpu.VMEM((NBMAX, BATCH), jnp.int32),
            pltpu.VMEM((BATCH, D), jnp.float32),
            pltpu.VMEM((BATCH, D), jnp.float32),
            pltpu.VMEM_SHARED((N_PAD, D), jnp.float32),
            pltpu.SemaphoreType.DMA,
            pltpu.SemaphoreType.DMA,
        ],
    )
    def sc_kern(f_hbm, src_hbm, et_hbm, dst_hbm, out_hbm,
                key_v, dst_v, rows0, rows1, acc, sem0, sem1):
        cid = lax.axis_index("c")
        sid = lax.axis_index("s")
        nb = jnp.where(cid == 0, NB0, NB1)
        bstart = jnp.where(cid == 0, sid * NB0, NS * NB0 + sid * NB1)

        @pl.when(cid == 0)
        def _stage0():
            pltpu.sync_copy(et_hbm.at[pl.ds(bstart, NB0)], key_v.at[pl.ds(0, NB0)])
            pltpu.sync_copy(src_hbm.at[pl.ds(bstart, NB0)], dst_v.at[pl.ds(0, NB0)])

        @pl.when(cid == 1)
        def _stage1():
            pltpu.sync_copy(et_hbm.at[pl.ds(bstart, NB1)], key_v.at[pl.ds(0, NB1)])
            pltpu.sync_copy(src_hbm.at[pl.ds(bstart, NB1)], dst_v.at[pl.ds(0, NB1)])

        def key_body(j, _):
            for c in range(BATCH // 16):
                sl = pl.ds(c * 16, 16)
                key_v[j, sl] = key_v[j, sl] * N_NODES + dst_v[j, sl]
            return _

        lax.fori_loop(0, nb, key_body, None)

        @pl.when(cid == 0)
        def _staged0():
            pltpu.sync_copy(dst_hbm.at[pl.ds(bstart, NB0)], dst_v.at[pl.ds(0, NB0)])

        @pl.when(cid == 1)
        def _staged1():
            pltpu.sync_copy(dst_hbm.at[pl.ds(bstart, NB1)], dst_v.at[pl.ds(0, NB1)])

        zero16 = jnp.zeros((16,), jnp.float32)

        def zero_body(i, _):
            for c in range(D // 16):
                rows0[i, pl.ds(c * 16, 16)] = zero16
            return _

        lax.fori_loop(0, BATCH, zero_body, None)
        for k in range(SLAB // BATCH):
            pltpu.sync_copy(rows0, acc.at[pl.ds(sid * SLAB + k * BATCH, BATCH)])
        rem = SLAB % BATCH
        pltpu.sync_copy(
            rows0.at[pl.ds(0, rem)],
            acc.at[pl.ds(sid * SLAB + (SLAB // BATCH) * BATCH, rem)])

        plsc.subcore_barrier()

        bufs = (rows0, rows1)
        sems = (sem0, sem1)

        def gstart(bidx, b):
            pltpu.async_copy(f_hbm.at[key_v.at[bidx]], bufs[b], sems[b])

        for b in range(NBUF):
            gstart(b, b)

        def pipe_body(j, _):
            for b in range(NBUF):
                idx = j * NBUF + b
                pltpu.make_async_copy(
                    f_hbm.at[pl.ds(0, BATCH)], bufs[b], sems[b]
                ).wait()
                pltpu.sync_copy(bufs[b], acc.at[dst_v.at[idx]], add=True)

                @pl.when(idx + NBUF < nb)
                def _start_next():
                    gstart(idx + NBUF, b)
            return _

        lax.fori_loop(0, nb // NBUF, pipe_body, None)

        plsc.subcore_barrier()
        pltpu.sync_copy(acc.at[pl.ds(sid * SLAB, SLAB)],
                        out_hbm.at[cid, pl.ds(sid * SLAB, SLAB)])

    return sc_kern(f_table, src_w, et_w, dst_w)


def kernel(feat, edge_index, etypes, weight):
    src = edge_index[0]
    dst = edge_index[1]
    pad = E_PAD - N_EDGES
    src_p = jnp.concatenate([src, jnp.zeros((pad,), jnp.int32)])
    et_p = jnp.concatenate([etypes, jnp.zeros((pad,), jnp.int32)])
    dst_p = jnp.concatenate(
        [dst, N_NODES + (jnp.arange(pad, dtype=jnp.int32) % (N_PAD - N_NODES))]
    )

    f_table = _rel_matmul(feat, weight).reshape(R * N_NODES, D)
    partials = _sc_gather_scatter(
        f_table,
        src_p.reshape(NBT, BATCH),
        et_p.reshape(NBT, BATCH),
        dst_p.reshape(NBT, BATCH),
    )
    out = _sum_partials(partials)
    return out[:N_NODES]
